# baseline (speedup 1.0000x reference)
import functools

import jax
import jax.numpy as jnp
from jax import lax
from jax.experimental import pallas as pl
from jax.experimental.pallas import tpu as pltpu
from jax.experimental.pallas import tpu_sc as plsc

NC = 2
NS = 16
NW = NC * NS
MAXLEN = 200
EMBED = 64
BATCH = 4096
LANES = 128
CHUNK = 128
BFLAT = BATCH * MAXLEN
NCHUNKS = BFLAT // CHUNK
CPW = NCHUNKS // NW
NBUF = 4
SEQ_BLK = 8


def _gather_body(x2_hbm, tab_hbm, out_hbm, idx_v,
                 b0, b1, b2, b3, g0, g1, g2, g3, o0, o1, o2, o3):
    bufs = (b0, b1, b2, b3)
    gsems = (g0, g1, g2, g3)
    osems = (o0, o1, o2, o3)

    wid = lax.axis_index("s") * NC + lax.axis_index("c")
    row0 = wid * CPW

    pltpu.sync_copy(x2_hbm.at[pl.ds(row0, CPW)], idx_v)

    def gstart(c, p):
        pltpu.make_async_copy(tab_hbm.at[idx_v.at[c]], bufs[p], gsems[p]).start()

    def gwait(c, p):
        pltpu.make_async_copy(tab_hbm.at[idx_v.at[c]], bufs[p], gsems[p]).wait()

    def odesc(c, p):
        dst = out_hbm.at[pl.ds((row0 + c) * CHUNK, CHUNK), pl.ds(0, EMBED)]
        return pltpu.make_async_copy(bufs[p], dst, osems[p])

    for p in range(NBUF - 1):
        gstart(p, p)

    @pl.loop(0, CPW, step=NBUF)
    def _(c):
        for k in range(NBUF):
            ck = c + k
            p = k
            pn = (k + NBUF - 1) % NBUF

            @pl.when(ck + NBUF - 1 < CPW)
            def _():
                @pl.when(ck >= 1)
                def _():
                    odesc(ck - 1, pn).wait()
                gstart(ck + NBUF - 1, pn)

            gwait(ck, p)
            odesc(ck, p).start()

    for k in range(NBUF):
        odesc(CPW - NBUF + k, k).wait()


def _addpos_body(tok_ref, pos_ref, out_ref):
    x = tok_ref[...]
    a = x[:, :EMBED]
    p = pos_ref[...]
    for s in range(SEQ_BLK):
        out_ref[s, :, :] = a[s * MAXLEN:(s + 1) * MAXLEN, :] + p


@jax.jit
def kernel(x, token_table, pos_table):
    x2 = x.reshape(NCHUNKS, CHUNK).astype(jnp.int32)

    mesh = plsc.VectorSubcoreMesh(core_axis_name="c", subcore_axis_name="s")
    gather = pl.kernel(
        _gather_body,
        out_type=jax.ShapeDtypeStruct((BFLAT, LANES), jnp.float32),
        mesh=mesh,
        scratch_types=(
            [pltpu.VMEM((CPW, CHUNK), jnp.int32)]
            + [pltpu.VMEM((CHUNK, EMBED), jnp.float32)] * NBUF
            + [pltpu.SemaphoreType.DMA] * (2 * NBUF)
        ),
        compiler_params=pltpu.CompilerParams(use_tc_tiling_on_sc=False),
    )
    tok128 = gather(x2, token_table)

    out = pl.pallas_call(
        _addpos_body,
        out_shape=jax.ShapeDtypeStruct((BATCH, MAXLEN, EMBED), jnp.float32),
        grid=(BATCH // SEQ_BLK,),
        in_specs=[
            pl.BlockSpec((SEQ_BLK * MAXLEN, LANES), lambda i: (i, 0)),
            pl.BlockSpec((MAXLEN, EMBED), lambda i: (0, 0)),
        ],
        out_specs=pl.BlockSpec((SEQ_BLK, MAXLEN, EMBED), lambda i: (i, 0, 0)),
        compiler_params=pltpu.CompilerParams(
            dimension_semantics=("parallel",)),
    )(tok128, pos_table)
    return out


# speedup vs baseline: 2.1046x; 2.1046x over previous
import functools

import jax
import jax.numpy as jnp
from jax import lax
from jax.experimental import pallas as pl
from jax.experimental.pallas import tpu as pltpu
from jax.experimental.pallas import tpu_sc as plsc

NC = 2
NS = 16
NW = NC * NS
MAXLEN = 200
EMBED = 64
BATCH = 4096
LANES = 128
CHUNK = 128
BFLAT = BATCH * MAXLEN
NCHUNKS = BFLAT // CHUNK
CPW = NCHUNKS // NW
NBUF = 5


def _emb_body(x2_hbm, tab_hbm, pos2_hbm, out_hbm, idx_v, pos_v,
              b0, b1, b2, b3, b4, g0, g1, g2, g3, g4, o0, o1, o2, o3, o4):
    bufs = (b0, b1, b2, b3, b4)
    gsems = (g0, g1, g2, g3, g4)
    osems = (o0, o1, o2, o3, o4)

    wid = lax.axis_index("s") * NC + lax.axis_index("c")
    row0 = wid * CPW

    pltpu.sync_copy(pos2_hbm, pos_v)
    pltpu.sync_copy(x2_hbm.at[pl.ds(row0, CPW)], idx_v)

    def gstart(c, p):
        pltpu.make_async_copy(tab_hbm.at[idx_v.at[c]], bufs[p], gsems[p]).start()

    def gwait(c, p):
        pltpu.make_async_copy(tab_hbm.at[idx_v.at[c]], bufs[p], gsems[p]).wait()

    def odesc(c, p):
        dst = out_hbm.at[pl.ds((row0 + c) * CHUNK, CHUNK), pl.ds(0, EMBED)]
        return pltpu.make_async_copy(bufs[p], dst, osems[p])

    for p in range(NBUF - 1):
        gstart(p, p)

    @pl.loop(0, CPW, step=NBUF)
    def _(c):
        for k in range(NBUF):
            ck = c + k
            p = k
            pn = (k + NBUF - 1) % NBUF

            @pl.when(ck + NBUF - 1 < CPW)
            def _():
                @pl.when(ck >= 1)
                def _():
                    odesc(ck - 1, pn).wait()
                gstart(ck + NBUF - 1, pn)

            gwait(ck, p)
            buf = bufs[p]
            off = lax.rem(ck * CHUNK, MAXLEN)

            @plsc.parallel_loop(0, CHUNK, unroll=4)
            def _(r):
                for g in range(EMBED // 16):
                    s = pl.ds(g * 16, 16)
                    plsc.addupdate(buf.at[r, s], pos_v[off + r, s])

            odesc(ck, p).start()

    for k in range(NBUF):
        odesc(CPW - NBUF + k, k).wait()


@jax.jit
def kernel(x, token_table, pos_table):
    x2 = x.reshape(NCHUNKS, CHUNK).astype(jnp.int32)
    pos2 = jnp.concatenate([pos_table, pos_table], axis=0)

    mesh = plsc.VectorSubcoreMesh(core_axis_name="c", subcore_axis_name="s")
    run = pl.kernel(
        _emb_body,
        out_type=jax.ShapeDtypeStruct((BFLAT, LANES), jnp.float32),
        mesh=mesh,
        scratch_types=(
            [pltpu.VMEM((CPW, CHUNK), jnp.int32),
             pltpu.VMEM((2 * MAXLEN, EMBED), jnp.float32)]
            + [pltpu.VMEM((CHUNK, EMBED), jnp.float32)] * NBUF
            + [pltpu.SemaphoreType.DMA] * (2 * NBUF)
        ),
        compiler_params=pltpu.CompilerParams(use_tc_tiling_on_sc=False),
    )
    out128 = run(x2, token_table, pos2)
    return out128[:, :EMBED].reshape(BATCH, MAXLEN, EMBED)


# speedup vs baseline: 2.1065x; 1.0009x over previous
import functools

import jax
import jax.numpy as jnp
from jax import lax
from jax.experimental import pallas as pl
from jax.experimental.pallas import tpu as pltpu
from jax.experimental.pallas import tpu_sc as plsc

NC = 2
NS = 16
NW = NC * NS
MAXLEN = 200
EMBED = 64
BATCH = 4096
LANES = 128
CHUNK = 128
BFLAT = BATCH * MAXLEN
NCHUNKS = BFLAT // CHUNK
CPW = NCHUNKS // NW
NBUF = 5


def _emb_body(x2_hbm, tab_hbm, pos2_hbm, out_hbm, idx_v, pos_v,
              b0, b1, b2, b3, b4, g0, g1, g2, g3, g4, o0, o1, o2, o3, o4):
    bufs = (b0, b1, b2, b3, b4)
    gsems = (g0, g1, g2, g3, g4)
    osems = (o0, o1, o2, o3, o4)

    wid = lax.axis_index("s") * NC + lax.axis_index("c")
    row0 = wid * CPW

    pltpu.sync_copy(pos2_hbm, pos_v)
    pltpu.sync_copy(x2_hbm.at[pl.ds(row0, CPW)], idx_v)

    def gstart(c, p):
        pltpu.make_async_copy(tab_hbm.at[idx_v.at[c]], bufs[p], gsems[p]).start()

    def gwait(c, p):
        pltpu.make_async_copy(tab_hbm.at[idx_v.at[c]], bufs[p], gsems[p]).wait()

    def odesc(c, p):
        dst = out_hbm.at[pl.ds((row0 + c) * CHUNK, CHUNK), pl.ds(0, EMBED)]
        return pltpu.make_async_copy(bufs[p], dst, osems[p])

    for p in range(NBUF - 1):
        gstart(p, p)

    @pl.loop(0, CPW, step=NBUF)
    def _(c):
        for k in range(NBUF):
            ck = c + k
            p = k
            pn = (k + NBUF - 1) % NBUF

            @pl.when(ck + NBUF - 1 < CPW)
            def _():
                @pl.when(ck >= 1)
                def _():
                    odesc(ck - 1, pn).wait()
                gstart(ck + NBUF - 1, pn)

            gwait(ck, p)
            buf = bufs[p]
            off = lax.rem(ck * CHUNK, MAXLEN)

            @plsc.parallel_loop(0, CHUNK, unroll=8)
            def _(r):
                for g in range(EMBED // 16):
                    s = pl.ds(g * 16, 16)
                    plsc.addupdate(buf.at[r, s], pos_v[off + r, s])

            odesc(ck, p).start()

    for k in range(NBUF):
        odesc(CPW - NBUF + k, k).wait()


@jax.jit
def kernel(x, token_table, pos_table):
    x2 = x.reshape(NCHUNKS, CHUNK).astype(jnp.int32)
    pos2 = jnp.concatenate([pos_table, pos_table], axis=0)

    mesh = plsc.VectorSubcoreMesh(core_axis_name="c", subcore_axis_name="s")
    run = pl.kernel(
        _emb_body,
        out_type=jax.ShapeDtypeStruct((BFLAT, LANES), jnp.float32),
        mesh=mesh,
        scratch_types=(
            [pltpu.VMEM((CPW, CHUNK), jnp.int32),
             pltpu.VMEM((2 * MAXLEN, EMBED), jnp.float32)]
            + [pltpu.VMEM((CHUNK, EMBED), jnp.float32)] * NBUF
            + [pltpu.SemaphoreType.DMA] * (2 * NBUF)
        ),
        compiler_params=pltpu.CompilerParams(use_tc_tiling_on_sc=False),
    )
    out128 = run(x2, token_table, pos2)
    return out128[:, :EMBED].reshape(BATCH, MAXLEN, EMBED)
